# baseline (speedup 1.0000x reference)
import functools

import jax
import jax.numpy as jnp
from jax import lax
from jax.experimental import pallas as pl
from jax.experimental.pallas import tpu as pltpu
from jax.experimental.pallas import tpu_sc as plsc

_NUM_EMBED = 1024
_LATENT_DIM = 64
_BETA = 0.25
_TN = 512


def _dist_argmin_body(z_ref, cb_ref, ids_ref, dsum_ref):
    z = z_ref[...]
    cb = cb_ref[...]
    s = lax.dot_general(z, cb, (((1,), (1,)), ((), ())),
                        preferred_element_type=jnp.float32)
    z2 = jnp.sum(z * z, axis=1, keepdims=True)
    c2 = jnp.sum(cb * cb, axis=1)[None, :]
    d = (z2 + c2) - 2.0 * s
    m = jnp.min(d, axis=1, keepdims=True)
    col = lax.broadcasted_iota(jnp.int32, d.shape, 1)
    ids_ref[...] = jnp.min(jnp.where(d == m, col, _NUM_EMBED), axis=1)

    @pl.when(pl.program_id(0) == 0)
    def _init():
        dsum_ref[0] = 0.0

    dsum_ref[0] += jnp.sum(m)


def _dist_argmin(z2d, codebook):
    n = z2d.shape[0]
    return pl.pallas_call(
        _dist_argmin_body,
        grid=(n // _TN,),
        in_specs=[
            pl.BlockSpec((_TN, _LATENT_DIM), lambda i: (i, 0)),
            pl.BlockSpec((_NUM_EMBED, _LATENT_DIM), lambda i: (0, 0)),
        ],
        out_specs=[
            pl.BlockSpec((_TN,), lambda i: (i,)),
            pl.BlockSpec(memory_space=pltpu.SMEM),
        ],
        out_shape=[
            jax.ShapeDtypeStruct((n,), jnp.int32),
            jax.ShapeDtypeStruct((1,), jnp.float32),
        ],
    )(z2d, codebook)


def _sc_gather(codebook, ids):
    info = plsc.get_sparse_core_info()
    nw = info.num_cores * info.num_subcores
    n = ids.shape[0]
    b_per_w = n // nw
    nchunk = b_per_w // 128
    mesh = plsc.VectorSubcoreMesh(core_axis_name="c", subcore_axis_name="s")

    @functools.partial(
        pl.kernel,
        mesh=mesh,
        compiler_params=pltpu.CompilerParams(use_tc_tiling_on_sc=False),
        out_type=jax.ShapeDtypeStruct((n, _LATENT_DIM), jnp.float32),
        scratch_types=[
            pltpu.VMEM((b_per_w,), jnp.int32),
            pltpu.VMEM((b_per_w, _LATENT_DIM), jnp.float32),
            pltpu.SemaphoreType.DMA,
        ],
    )
    def gather_k(table_hbm, idx_hbm, out_hbm, idx_v, rows_v, sem):
        wid = lax.axis_index("s") * info.num_cores + lax.axis_index("c")
        base = wid * b_per_w
        pltpu.sync_copy(idx_hbm.at[pl.ds(base, b_per_w)], idx_v)
        copies = [
            pltpu.async_copy(
                table_hbm.at[idx_v.at[pl.ds(j * 128, 128)]],
                rows_v.at[pl.ds(j * 128, 128)],
                sem,
            )
            for j in range(nchunk)
        ]
        for c in copies:
            c.wait()
        pltpu.sync_copy(rows_v, out_hbm.at[pl.ds(base, b_per_w)])

    return gather_k(codebook, ids)


def kernel(z, codebook):
    z2d = z.reshape(-1, _LATENT_DIM)
    n = z2d.shape[0]
    ids, dsum = _dist_argmin(z2d, codebook)
    z_q = _sc_gather(codebook, ids).reshape(z.shape)
    loss = (1.0 + _BETA) * (dsum[0] / (n * _LATENT_DIM))
    return (z, z_q, ids, loss)


# speedup vs baseline: 1.5705x; 1.5705x over previous
import functools

import jax
import jax.numpy as jnp
from jax import lax
from jax.experimental import pallas as pl
from jax.experimental.pallas import tpu as pltpu
from jax.experimental.pallas import tpu_sc as plsc

_NUM_EMBED = 1024
_LATENT_DIM = 64
_BETA = 0.25
_TN = 1024


def _dist_argmin_body(z_ref, cb_ref, ids_ref, dsum_ref, cbneg2_ref, c2_ref):
    @pl.when(pl.program_id(0) == 0)
    def _prep():
        cb = cb_ref[...]
        cbneg2_ref[...] = -2.0 * cb
        c2_ref[...] = jnp.sum(cb * cb, axis=1, keepdims=True)
        dsum_ref[0] = 0.0

    z = z_ref[...]
    d = lax.dot_general(cbneg2_ref[...], z, (((1,), (1,)), ((), ())),
                        preferred_element_type=jnp.float32)
    c2 = c2_ref[...]
    rv = d[0:8, :] + c2[0:8, :]
    ri = jnp.zeros((8, _TN), jnp.float32)
    for r in range(1, _NUM_EMBED // 8):
        v = d[8 * r:8 * r + 8, :] + c2[8 * r:8 * r + 8, :]
        cm = v < rv
        rv = jnp.where(cm, v, rv)
        ri = jnp.where(cm, float(r), ri)
    srow = lax.broadcasted_iota(jnp.int32, (8, _TN), 0).astype(jnp.float32)
    code = ri * 8.0 + srow
    m1 = jnp.min(rv, axis=0, keepdims=True)
    idsf = jnp.min(jnp.where(rv == m1, code, float(_NUM_EMBED)), axis=0)
    ids_ref[...] = idsf.astype(jnp.int32)
    dsum_ref[0] += jnp.sum(m1) + jnp.sum(z * z)


def _dist_argmin(z2d, codebook):
    n = z2d.shape[0]
    return pl.pallas_call(
        _dist_argmin_body,
        grid=(n // _TN,),
        in_specs=[
            pl.BlockSpec((_TN, _LATENT_DIM), lambda i: (i, 0)),
            pl.BlockSpec((_NUM_EMBED, _LATENT_DIM), lambda i: (0, 0)),
        ],
        out_specs=[
            pl.BlockSpec((_TN,), lambda i: (i,)),
            pl.BlockSpec(memory_space=pltpu.SMEM),
        ],
        out_shape=[
            jax.ShapeDtypeStruct((n,), jnp.int32),
            jax.ShapeDtypeStruct((1,), jnp.float32),
        ],
        scratch_shapes=[
            pltpu.VMEM((_NUM_EMBED, _LATENT_DIM), jnp.float32),
            pltpu.VMEM((_NUM_EMBED, 1), jnp.float32),
        ],
    )(z2d, codebook)


def _sc_gather(codebook, ids):
    info = plsc.get_sparse_core_info()
    nw = info.num_cores * info.num_subcores
    n = ids.shape[0]
    b_per_w = n // nw
    nchunk = b_per_w // 128
    mesh = plsc.VectorSubcoreMesh(core_axis_name="c", subcore_axis_name="s")

    @functools.partial(
        pl.kernel,
        mesh=mesh,
        compiler_params=pltpu.CompilerParams(use_tc_tiling_on_sc=False),
        out_type=jax.ShapeDtypeStruct((n, _LATENT_DIM), jnp.float32),
        scratch_types=[
            pltpu.VMEM((b_per_w,), jnp.int32),
            pltpu.VMEM((b_per_w, _LATENT_DIM), jnp.float32),
            pltpu.SemaphoreType.DMA,
        ],
    )
    def gather_k(table_hbm, idx_hbm, out_hbm, idx_v, rows_v, sem):
        wid = lax.axis_index("s") * info.num_cores + lax.axis_index("c")
        base = wid * b_per_w
        pltpu.sync_copy(idx_hbm.at[pl.ds(base, b_per_w)], idx_v)
        copies = [
            pltpu.async_copy(
                table_hbm.at[idx_v.at[pl.ds(j * 128, 128)]],
                rows_v.at[pl.ds(j * 128, 128)],
                sem,
            )
            for j in range(nchunk)
        ]
        for c in copies:
            c.wait()
        pltpu.sync_copy(rows_v, out_hbm.at[pl.ds(base, b_per_w)])

    return gather_k(codebook, ids)


def kernel(z, codebook):
    z2d = z.reshape(-1, _LATENT_DIM)
    n = z2d.shape[0]
    ids, dsum = _dist_argmin(z2d, codebook)
    z_q = _sc_gather(codebook, ids).reshape(z.shape)
    loss = (1.0 + _BETA) * (dsum[0] / (n * _LATENT_DIM))
    return (z, z_q, ids, loss)
